# Initial kernel scaffold; baseline (speedup 1.0000x reference)
import functools

import jax
import jax.numpy as jnp
from jax import lax
from jax.experimental import pallas as pl
from jax.experimental.pallas import tpu as pltpu
from jax.experimental.pallas import tpu_sc as plsc

NC = 2
NS = 16
L = 16
NW = NC * NS
CHUNK = 128


def _deg_kernel(n_pad, chunks_per_tile):
    own = n_pad // NS
    mesh = plsc.VectorSubcoreMesh(core_axis_name="c", subcore_axis_name="s",
                                  num_cores=NC, num_subcores=NS)

    @functools.partial(
        pl.kernel,
        out_type=jax.ShapeDtypeStruct((NC, n_pad, L), jnp.float32),
        mesh=mesh,
        scratch_types=[
            pltpu.VMEM((chunks_per_tile, CHUNK), jnp.int32),
            pltpu.VMEM((CHUNK, L), jnp.float32),
            pltpu.VMEM_SHARED((n_pad, L), jnp.float32),
        ],
    )
    def deg_k(edges_hbm, ones_hbm, zeros_hbm, out_hbm, idx_v, ones_v, deg_sh):
        c = lax.axis_index("c")
        s = lax.axis_index("s")
        pltpu.sync_copy(ones_hbm, ones_v)
        pltpu.sync_copy(
            edges_hbm.at[c, pl.ds(s * chunks_per_tile, chunks_per_tile)],
            idx_v)
        pltpu.sync_copy(zeros_hbm, deg_sh.at[pl.ds(s * own, own)])
        plsc.subcore_barrier()

        def body(j, carry):
            pltpu.sync_copy(ones_v, deg_sh.at[idx_v.at[j]], add=True)
            return carry

        lax.fori_loop(0, chunks_per_tile, body, 0)
        plsc.subcore_barrier()
        pltpu.sync_copy(deg_sh.at[pl.ds(s * own, own)],
                        out_hbm.at[c, pl.ds(s * own, own)])

    return deg_k


def _agg_kernel(n_pad, d, chunks_per_tile):
    own = n_pad // NS
    mesh = plsc.VectorSubcoreMesh(core_axis_name="c", subcore_axis_name="s",
                                  num_cores=NC, num_subcores=NS)

    @functools.partial(
        pl.kernel,
        out_type=jax.ShapeDtypeStruct((NC, n_pad, d), jnp.float32),
        mesh=mesh,
        scratch_types=[
            pltpu.VMEM((chunks_per_tile, CHUNK), jnp.int32),
            pltpu.VMEM((chunks_per_tile, CHUNK), jnp.int32),
            pltpu.VMEM((CHUNK, d), jnp.float32),
            pltpu.VMEM_SHARED((n_pad, d), jnp.float32),
            pltpu.SemaphoreType.DMA,
        ],
    )
    def agg_k(edges_hbm, h_hbm, zeros_hbm, out_hbm,
              src_v, dst_v, rows_v, agg_sh, sem):
        c = lax.axis_index("c")
        s = lax.axis_index("s")
        w = c * NS + s
        pltpu.sync_copy(edges_hbm.at[0, pl.ds(w * chunks_per_tile,
                                              chunks_per_tile)], src_v)
        pltpu.sync_copy(edges_hbm.at[1, pl.ds(w * chunks_per_tile,
                                              chunks_per_tile)], dst_v)
        pltpu.sync_copy(zeros_hbm, agg_sh.at[pl.ds(s * own, own)])
        plsc.subcore_barrier()

        def body(j, carry):
            pltpu.async_copy(h_hbm.at[src_v.at[j]], rows_v, sem).wait()
            pltpu.sync_copy(rows_v, agg_sh.at[dst_v.at[j]], add=True)
            return carry

        lax.fori_loop(0, chunks_per_tile, body, 0)
        plsc.subcore_barrier()
        pltpu.sync_copy(agg_sh.at[pl.ds(s * own, own)],
                        out_hbm.at[c, pl.ds(s * own, own)])

    return agg_k


def _scale_call(deg0, feats_pad, n_pad, d, blk):
    def k2(dg_ref, ft_ref, out_ref):
        norm = lax.rsqrt(jnp.maximum(dg_ref[...], 1.0))
        out_ref[...] = ft_ref[...] * norm

    return pl.pallas_call(
        k2,
        grid=(n_pad // blk,),
        in_specs=[pl.BlockSpec((blk, 1), lambda i: (i, 0)),
                  pl.BlockSpec((blk, d), lambda i: (i, 0))],
        out_specs=pl.BlockSpec((blk, d), lambda i: (i, 0)),
        out_shape=jax.ShapeDtypeStruct((n_pad, d), jnp.float32),
    )(deg0, feats_pad)


def _linear_call(p0, p1, deg1, feats_pad, W, b2, n, n_pad, d, blk):
    def k4a(p0_ref, p1_ref, dg_ref, ft_ref, w_ref, b_ref, h_ref, st_ref):
        i = pl.program_id(0)
        nd = lax.rsqrt(jnp.maximum(dg_ref[...], 1.0))
        agg = (p0_ref[...] + p1_ref[...]) * nd
        h = jnp.dot(agg, w_ref[...], preferred_element_type=jnp.float32)
        h = h + b_ref[...] + ft_ref[...]
        h_ref[...] = h
        rows = i * blk + lax.broadcasted_iota(jnp.int32, (blk, 1), 0)
        hm = jnp.where(rows < n, h, 0.0)
        s1 = jnp.sum(hm, axis=0, keepdims=True)
        s2 = jnp.sum(hm * hm, axis=0, keepdims=True)
        st = jnp.concatenate([s1, s2], axis=0)

        @pl.when(i == 0)
        def _():
            st_ref[...] = st

        @pl.when(i > 0)
        def _():
            st_ref[...] = st_ref[...] + st

    return pl.pallas_call(
        k4a,
        grid=(n_pad // blk,),
        in_specs=[pl.BlockSpec((blk, d), lambda i: (i, 0)),
                  pl.BlockSpec((blk, d), lambda i: (i, 0)),
                  pl.BlockSpec((blk, 1), lambda i: (i, 0)),
                  pl.BlockSpec((blk, d), lambda i: (i, 0)),
                  pl.BlockSpec((d, d), lambda i: (0, 0)),
                  pl.BlockSpec((1, d), lambda i: (0, 0))],
        out_specs=[pl.BlockSpec((blk, d), lambda i: (i, 0)),
                   pl.BlockSpec((2, d), lambda i: (0, 0))],
        out_shape=[jax.ShapeDtypeStruct((n_pad, d), jnp.float32),
                   jax.ShapeDtypeStruct((2, d), jnp.float32)],
    )(p0, p1, deg1, feats_pad, W, b2)


def _bn_call(h, stats, gamma2, beta2, n, n_pad, d, blk):
    def k4b(h_ref, st_ref, g_ref, bt_ref, out_ref):
        inv_n = 1.0 / n
        mean = st_ref[0:1, :] * inv_n
        var = st_ref[1:2, :] * inv_n - mean * mean
        inv = lax.rsqrt(var + 1e-5)
        out_ref[...] = (h_ref[...] - mean) * (inv * g_ref[...]) + bt_ref[...]

    return pl.pallas_call(
        k4b,
        grid=(n_pad // blk,),
        in_specs=[pl.BlockSpec((blk, d), lambda i: (i, 0)),
                  pl.BlockSpec((2, d), lambda i: (0, 0)),
                  pl.BlockSpec((1, d), lambda i: (0, 0)),
                  pl.BlockSpec((1, d), lambda i: (0, 0))],
        out_specs=pl.BlockSpec((blk, d), lambda i: (i, 0)),
        out_shape=jax.ShapeDtypeStruct((n_pad, d), jnp.float32),
    )(h, stats, gamma2, beta2)


@jax.jit
def kernel(feats, edge_index, W, b, gamma, beta):
    n, d = feats.shape
    e = edge_index.shape[1]

    e_unit = NW * CHUNK * 8
    e_pad = ((e + e_unit - 1) // e_unit) * e_unit
    n_unit = NS * CHUNK
    n_pad = ((n + 1 + n_unit - 1) // n_unit) * n_unit

    edges = jnp.concatenate(
        [edge_index,
         jnp.full((2, e_pad - e), n, dtype=jnp.int32)], axis=1)
    edges = edges.reshape(2, e_pad // CHUNK, CHUNK)

    ones_rows = jnp.ones((CHUNK, L), dtype=jnp.float32)
    zeros_deg = jnp.zeros((n_pad // NS, L), dtype=jnp.float32)
    zeros_agg = jnp.zeros((n_pad // NS, d), dtype=jnp.float32)
    feats_pad = jnp.concatenate(
        [feats, jnp.zeros((n_pad - n, d), dtype=feats.dtype)], axis=0)

    k1_chunks = e_pad // (NS * CHUNK)
    k3_chunks = e_pad // (NW * CHUNK)

    deg = _deg_kernel(n_pad, k1_chunks)(edges, ones_rows, zeros_deg)
    deg0 = deg[0, :, 0:1]
    deg1 = deg[1, :, 0:1]

    blk = 1024
    h_scaled = _scale_call(deg0, feats_pad, n_pad, d, blk)

    parts = _agg_kernel(n_pad, d, k3_chunks)(edges, h_scaled, zeros_agg)

    b2 = b.reshape(1, d)
    gamma2 = gamma.reshape(1, d)
    beta2 = beta.reshape(1, d)
    h, stats = _linear_call(parts[0], parts[1], deg1, feats_pad, W, b2,
                            n, n_pad, d, blk)
    out = _bn_call(h, stats, gamma2, beta2, n, n_pad, d, blk)
    return out[:n]


# speedup vs baseline: 3.3484x; 3.3484x over previous
import functools

import jax
import jax.numpy as jnp
from jax import lax
from jax.experimental import pallas as pl
from jax.experimental.pallas import tpu as pltpu
from jax.experimental.pallas import tpu_sc as plsc

NC = 2
NS = 16
L = 16
NW = NC * NS
CHUNK = 128


def _deg_kernel(n_pad, d, chunks_per_tile):
    own = n_pad // NS
    mesh = plsc.VectorSubcoreMesh(core_axis_name="c", subcore_axis_name="s",
                                  num_cores=NC, num_subcores=NS)

    @functools.partial(
        pl.kernel,
        out_type=jax.ShapeDtypeStruct((NC, n_pad, d), jnp.float32),
        mesh=mesh,
        scratch_types=[
            pltpu.VMEM((chunks_per_tile, CHUNK), jnp.int32),
            pltpu.VMEM((CHUNK, d), jnp.float32),
            pltpu.VMEM_SHARED((n_pad, d), jnp.float32),
        ],
    )
    def deg_k(edges_hbm, ones_hbm, zeros_hbm, out_hbm, idx_v, ones_v, deg_sh):
        c = lax.axis_index("c")
        s = lax.axis_index("s")
        pltpu.sync_copy(ones_hbm, ones_v)
        pltpu.sync_copy(
            edges_hbm.at[c, pl.ds(s * chunks_per_tile, chunks_per_tile)],
            idx_v)
        pltpu.sync_copy(zeros_hbm, deg_sh.at[pl.ds(s * own, own)])
        plsc.subcore_barrier()

        def body(j, carry):
            pltpu.sync_copy(ones_v, deg_sh.at[idx_v.at[j]], add=True)
            return carry

        lax.fori_loop(0, chunks_per_tile, body, 0)
        plsc.subcore_barrier()
        pltpu.sync_copy(deg_sh.at[pl.ds(s * own, own)],
                        out_hbm.at[c, pl.ds(s * own, own)])

    return deg_k


def _agg_kernel(n_pad, d, chunks_per_tile):
    own = n_pad // NS
    mesh = plsc.VectorSubcoreMesh(core_axis_name="c", subcore_axis_name="s",
                                  num_cores=NC, num_subcores=NS)

    @functools.partial(
        pl.kernel,
        out_type=jax.ShapeDtypeStruct((NC, n_pad, d), jnp.float32),
        mesh=mesh,
        scratch_types=[
            pltpu.VMEM((chunks_per_tile, CHUNK), jnp.int32),
            pltpu.VMEM((chunks_per_tile, CHUNK), jnp.int32),
            pltpu.VMEM((CHUNK, d), jnp.float32),
            pltpu.VMEM_SHARED((n_pad, d), jnp.float32),
            pltpu.SemaphoreType.DMA,
        ],
    )
    def agg_k(edges_hbm, h_hbm, zeros_hbm, out_hbm,
              src_v, dst_v, rows_v, agg_sh, sem):
        c = lax.axis_index("c")
        s = lax.axis_index("s")
        w = c * NS + s
        pltpu.sync_copy(edges_hbm.at[0, pl.ds(w * chunks_per_tile,
                                              chunks_per_tile)], src_v)
        pltpu.sync_copy(edges_hbm.at[1, pl.ds(w * chunks_per_tile,
                                              chunks_per_tile)], dst_v)
        pltpu.sync_copy(zeros_hbm, agg_sh.at[pl.ds(s * own, own)])
        plsc.subcore_barrier()

        def body(j, carry):
            pltpu.async_copy(h_hbm.at[src_v.at[j]], rows_v, sem).wait()
            pltpu.sync_copy(rows_v, agg_sh.at[dst_v.at[j]], add=True)
            return carry

        lax.fori_loop(0, chunks_per_tile, body, 0)
        plsc.subcore_barrier()
        pltpu.sync_copy(agg_sh.at[pl.ds(s * own, own)],
                        out_hbm.at[c, pl.ds(s * own, own)])

    return agg_k


def _scale_call(deg0, feats_pad, n_pad, d, blk):
    def k2(dg_ref, ft_ref, out_ref):
        norm = lax.rsqrt(jnp.maximum(dg_ref[...], 1.0))
        out_ref[...] = ft_ref[...] * norm

    return pl.pallas_call(
        k2,
        grid=(n_pad // blk,),
        in_specs=[pl.BlockSpec((blk, 1), lambda i: (i, 0)),
                  pl.BlockSpec((blk, d), lambda i: (i, 0))],
        out_specs=pl.BlockSpec((blk, d), lambda i: (i, 0)),
        out_shape=jax.ShapeDtypeStruct((n_pad, d), jnp.float32),
    )(deg0, feats_pad)


def _linear_call(p0, p1, deg1, feats_pad, W, b2, n, n_pad, d, blk):
    def k4a(p0_ref, p1_ref, dg_ref, ft_ref, w_ref, b_ref, h_ref, st_ref):
        i = pl.program_id(0)
        nd = lax.rsqrt(jnp.maximum(dg_ref[...], 1.0))
        agg = (p0_ref[...] + p1_ref[...]) * nd
        h = jnp.dot(agg, w_ref[...], preferred_element_type=jnp.float32)
        h = h + b_ref[...] + ft_ref[...]
        h_ref[...] = h
        rows = i * blk + lax.broadcasted_iota(jnp.int32, (blk, 1), 0)
        hm = jnp.where(rows < n, h, 0.0)
        s1 = jnp.sum(hm, axis=0, keepdims=True)
        s2 = jnp.sum(hm * hm, axis=0, keepdims=True)
        st = jnp.concatenate([s1, s2], axis=0)

        @pl.when(i == 0)
        def _():
            st_ref[...] = st

        @pl.when(i > 0)
        def _():
            st_ref[...] = st_ref[...] + st

    return pl.pallas_call(
        k4a,
        grid=(n_pad // blk,),
        in_specs=[pl.BlockSpec((blk, d), lambda i: (i, 0)),
                  pl.BlockSpec((blk, d), lambda i: (i, 0)),
                  pl.BlockSpec((blk, 1), lambda i: (i, 0)),
                  pl.BlockSpec((blk, d), lambda i: (i, 0)),
                  pl.BlockSpec((d, d), lambda i: (0, 0)),
                  pl.BlockSpec((1, d), lambda i: (0, 0))],
        out_specs=[pl.BlockSpec((blk, d), lambda i: (i, 0)),
                   pl.BlockSpec((2, d), lambda i: (0, 0))],
        out_shape=[jax.ShapeDtypeStruct((n_pad, d), jnp.float32),
                   jax.ShapeDtypeStruct((2, d), jnp.float32)],
    )(p0, p1, deg1, feats_pad, W, b2)


def _bn_call(h, stats, gamma2, beta2, n, n_pad, d, blk):
    def k4b(h_ref, st_ref, g_ref, bt_ref, out_ref):
        inv_n = 1.0 / n
        mean = st_ref[0:1, :] * inv_n
        var = st_ref[1:2, :] * inv_n - mean * mean
        inv = lax.rsqrt(var + 1e-5)
        out_ref[...] = (h_ref[...] - mean) * (inv * g_ref[...]) + bt_ref[...]

    return pl.pallas_call(
        k4b,
        grid=(n_pad // blk,),
        in_specs=[pl.BlockSpec((blk, d), lambda i: (i, 0)),
                  pl.BlockSpec((2, d), lambda i: (0, 0)),
                  pl.BlockSpec((1, d), lambda i: (0, 0)),
                  pl.BlockSpec((1, d), lambda i: (0, 0))],
        out_specs=pl.BlockSpec((blk, d), lambda i: (i, 0)),
        out_shape=jax.ShapeDtypeStruct((n_pad, d), jnp.float32),
    )(h, stats, gamma2, beta2)


@jax.jit
def kernel(feats, edge_index, W, b, gamma, beta):
    n, d = feats.shape
    e = edge_index.shape[1]

    e_unit = NW * CHUNK * 8
    e_pad = ((e + e_unit - 1) // e_unit) * e_unit
    n_unit = NS * CHUNK
    n_pad = ((n + 1 + n_unit - 1) // n_unit) * n_unit

    edges = jnp.concatenate(
        [edge_index,
         jnp.full((2, e_pad - e), n, dtype=jnp.int32)], axis=1)
    edges = edges.reshape(2, e_pad // CHUNK, CHUNK)

    ones_rows = jnp.ones((CHUNK, d), dtype=jnp.float32)
    zeros_agg = jnp.zeros((n_pad // NS, d), dtype=jnp.float32)
    feats_pad = jnp.concatenate(
        [feats, jnp.zeros((n_pad - n, d), dtype=feats.dtype)], axis=0)

    k1_chunks = e_pad // (NS * CHUNK)
    k3_chunks = e_pad // (NW * CHUNK)

    deg = _deg_kernel(n_pad, d, k1_chunks)(edges, ones_rows, zeros_agg)
    deg0 = deg[0, :, 0:1]
    deg1 = deg[1, :, 0:1]

    blk = 1024
    h_scaled = _scale_call(deg0, feats_pad, n_pad, d, blk)

    parts = _agg_kernel(n_pad, d, k3_chunks)(edges, h_scaled, zeros_agg)

    b2 = b.reshape(1, d)
    gamma2 = gamma.reshape(1, d)
    beta2 = beta.reshape(1, d)
    h, stats = _linear_call(parts[0], parts[1], deg1, feats_pad, W, b2,
                            n, n_pad, d, blk)
    out = _bn_call(h, stats, gamma2, beta2, n, n_pad, d, blk)
    return out[:n]


# speedup vs baseline: 6.3909x; 1.9086x over previous
import functools

import jax
import jax.numpy as jnp
from jax import lax
from jax.experimental import pallas as pl
from jax.experimental.pallas import tpu as pltpu
from jax.experimental.pallas import tpu_sc as plsc

NC = 2
NS = 16
L = 16
NW = NC * NS
CHUNK = 128


def _deg_kernel(n_pad, d, chunks_per_tile):
    own = n_pad // NS
    mesh = plsc.VectorSubcoreMesh(core_axis_name="c", subcore_axis_name="s",
                                  num_cores=NC, num_subcores=NS)

    @functools.partial(
        pl.kernel,
        out_type=jax.ShapeDtypeStruct((NC, n_pad, d), jnp.float32),
        mesh=mesh,
        scratch_types=[
            pltpu.VMEM((chunks_per_tile, CHUNK), jnp.int32),
            pltpu.VMEM((CHUNK, d), jnp.float32),
            pltpu.VMEM_SHARED((n_pad, d), jnp.float32),
            pltpu.SemaphoreType.DMA,
        ],
    )
    def deg_k(edges_hbm, ones_hbm, zeros_hbm, out_hbm, idx_v, ones_v, deg_sh,
              sem):
        c = lax.axis_index("c")
        s = lax.axis_index("s")
        pltpu.sync_copy(ones_hbm, ones_v)
        pltpu.sync_copy(
            edges_hbm.at[c, pl.ds(s * chunks_per_tile, chunks_per_tile)],
            idx_v)
        pltpu.sync_copy(zeros_hbm, deg_sh.at[pl.ds(s * own, own)])
        plsc.subcore_barrier()

        def fire(j, carry):
            pltpu.async_copy(ones_v, deg_sh.at[idx_v.at[j]], sem, add=True)
            return carry

        lax.fori_loop(0, chunks_per_tile, fire, 0)

        def drain(j, carry):
            pltpu.make_async_copy(ones_v, deg_sh.at[idx_v.at[j]], sem).wait()
            return carry

        lax.fori_loop(0, chunks_per_tile, drain, 0)
        plsc.subcore_barrier()
        pltpu.sync_copy(deg_sh.at[pl.ds(s * own, own)],
                        out_hbm.at[c, pl.ds(s * own, own)])

    return deg_k


def _agg_kernel(n_pad, d, chunks_per_tile):
    own = n_pad // NS
    mesh = plsc.VectorSubcoreMesh(core_axis_name="c", subcore_axis_name="s",
                                  num_cores=NC, num_subcores=NS)

    @functools.partial(
        pl.kernel,
        out_type=jax.ShapeDtypeStruct((NC, n_pad, d), jnp.float32),
        mesh=mesh,
        scratch_types=[
            pltpu.VMEM((chunks_per_tile, CHUNK), jnp.int32),
            pltpu.VMEM((chunks_per_tile, CHUNK), jnp.int32),
            pltpu.VMEM((CHUNK, d), jnp.float32),
            pltpu.VMEM_SHARED((n_pad, d), jnp.float32),
            pltpu.SemaphoreType.DMA,
        ],
    )
    def agg_k(edges_hbm, h_hbm, zeros_hbm, out_hbm,
              src_v, dst_v, rows_v, agg_sh, sem):
        c = lax.axis_index("c")
        s = lax.axis_index("s")
        w = c * NS + s
        pltpu.sync_copy(edges_hbm.at[0, pl.ds(w * chunks_per_tile,
                                              chunks_per_tile)], src_v)
        pltpu.sync_copy(edges_hbm.at[1, pl.ds(w * chunks_per_tile,
                                              chunks_per_tile)], dst_v)
        pltpu.sync_copy(zeros_hbm, agg_sh.at[pl.ds(s * own, own)])
        plsc.subcore_barrier()

        def body(j, carry):
            pltpu.async_copy(h_hbm.at[src_v.at[j]], rows_v, sem).wait()
            pltpu.sync_copy(rows_v, agg_sh.at[dst_v.at[j]], add=True)
            return carry

        lax.fori_loop(0, chunks_per_tile, body, 0)
        plsc.subcore_barrier()
        pltpu.sync_copy(agg_sh.at[pl.ds(s * own, own)],
                        out_hbm.at[c, pl.ds(s * own, own)])

    return agg_k


def _scale_call(deg0, feats_pad, n_pad, d, blk):
    def k2(dg_ref, ft_ref, out_ref):
        norm = lax.rsqrt(jnp.maximum(dg_ref[...], 1.0))
        out_ref[...] = ft_ref[...] * norm

    return pl.pallas_call(
        k2,
        grid=(n_pad // blk,),
        in_specs=[pl.BlockSpec((blk, 1), lambda i: (i, 0)),
                  pl.BlockSpec((blk, d), lambda i: (i, 0))],
        out_specs=pl.BlockSpec((blk, d), lambda i: (i, 0)),
        out_shape=jax.ShapeDtypeStruct((n_pad, d), jnp.float32),
    )(deg0, feats_pad)


def _linear_call(p0, p1, deg1, feats_pad, W, b2, n, n_pad, d, blk):
    def k4a(p0_ref, p1_ref, dg_ref, ft_ref, w_ref, b_ref, h_ref, st_ref):
        i = pl.program_id(0)
        nd = lax.rsqrt(jnp.maximum(dg_ref[...], 1.0))
        agg = (p0_ref[...] + p1_ref[...]) * nd
        h = jnp.dot(agg, w_ref[...], preferred_element_type=jnp.float32)
        h = h + b_ref[...] + ft_ref[...]
        h_ref[...] = h
        rows = i * blk + lax.broadcasted_iota(jnp.int32, (blk, 1), 0)
        hm = jnp.where(rows < n, h, 0.0)
        s1 = jnp.sum(hm, axis=0, keepdims=True)
        s2 = jnp.sum(hm * hm, axis=0, keepdims=True)
        st = jnp.concatenate([s1, s2], axis=0)

        @pl.when(i == 0)
        def _():
            st_ref[...] = st

        @pl.when(i > 0)
        def _():
            st_ref[...] = st_ref[...] + st

    return pl.pallas_call(
        k4a,
        grid=(n_pad // blk,),
        in_specs=[pl.BlockSpec((blk, d), lambda i: (i, 0)),
                  pl.BlockSpec((blk, d), lambda i: (i, 0)),
                  pl.BlockSpec((blk, 1), lambda i: (i, 0)),
                  pl.BlockSpec((blk, d), lambda i: (i, 0)),
                  pl.BlockSpec((d, d), lambda i: (0, 0)),
                  pl.BlockSpec((1, d), lambda i: (0, 0))],
        out_specs=[pl.BlockSpec((blk, d), lambda i: (i, 0)),
                   pl.BlockSpec((2, d), lambda i: (0, 0))],
        out_shape=[jax.ShapeDtypeStruct((n_pad, d), jnp.float32),
                   jax.ShapeDtypeStruct((2, d), jnp.float32)],
    )(p0, p1, deg1, feats_pad, W, b2)


def _bn_call(h, stats, gamma2, beta2, n, n_pad, d, blk):
    def k4b(h_ref, st_ref, g_ref, bt_ref, out_ref):
        inv_n = 1.0 / n
        mean = st_ref[0:1, :] * inv_n
        var = st_ref[1:2, :] * inv_n - mean * mean
        inv = lax.rsqrt(var + 1e-5)
        out_ref[...] = (h_ref[...] - mean) * (inv * g_ref[...]) + bt_ref[...]

    return pl.pallas_call(
        k4b,
        grid=(n_pad // blk,),
        in_specs=[pl.BlockSpec((blk, d), lambda i: (i, 0)),
                  pl.BlockSpec((2, d), lambda i: (0, 0)),
                  pl.BlockSpec((1, d), lambda i: (0, 0)),
                  pl.BlockSpec((1, d), lambda i: (0, 0))],
        out_specs=pl.BlockSpec((blk, d), lambda i: (i, 0)),
        out_shape=jax.ShapeDtypeStruct((n_pad, d), jnp.float32),
    )(h, stats, gamma2, beta2)


@jax.jit
def kernel(feats, edge_index, W, b, gamma, beta):
    n, d = feats.shape
    e = edge_index.shape[1]

    e_unit = NW * CHUNK * 8
    e_pad = ((e + e_unit - 1) // e_unit) * e_unit
    n_unit = NS * CHUNK
    n_pad = ((n + 1 + n_unit - 1) // n_unit) * n_unit

    sent = n + jnp.arange(e_pad - e, dtype=jnp.int32) % (n_pad - n)
    edges = jnp.concatenate(
        [edge_index, jnp.stack([sent, sent])], axis=1)
    edges = edges.reshape(2, e_pad // CHUNK, CHUNK)

    ones_rows = jnp.ones((CHUNK, d), dtype=jnp.float32)
    zeros_agg = jnp.zeros((n_pad // NS, d), dtype=jnp.float32)
    feats_pad = jnp.concatenate(
        [feats, jnp.zeros((n_pad - n, d), dtype=feats.dtype)], axis=0)

    k1_chunks = e_pad // (NS * CHUNK)
    k3_chunks = e_pad // (NW * CHUNK)

    deg = _deg_kernel(n_pad, d, k1_chunks)(edges, ones_rows, zeros_agg)
    deg0 = deg[0, :, 0:1]
    deg1 = deg[1, :, 0:1]

    blk = 1024
    h_scaled = _scale_call(deg0, feats_pad, n_pad, d, blk)

    parts = _agg_kernel(n_pad, d, k3_chunks)(edges, h_scaled, zeros_agg)

    b2 = b.reshape(1, d)
    gamma2 = gamma.reshape(1, d)
    beta2 = beta.reshape(1, d)
    h, stats = _linear_call(parts[0], parts[1], deg1, feats_pad, W, b2,
                            n, n_pad, d, blk)
    out = _bn_call(h, stats, gamma2, beta2, n, n_pad, d, blk)
    return out[:n]


# speedup vs baseline: 7.9940x; 1.2508x over previous
import functools

import jax
import jax.numpy as jnp
from jax import lax
from jax.experimental import pallas as pl
from jax.experimental.pallas import tpu as pltpu
from jax.experimental.pallas import tpu_sc as plsc

NC = 2
NS = 16
L = 16
NW = NC * NS
CHUNK = 128


def _deg_kernel(n_pad, d, chunks_per_tile):
    own = n_pad // NS
    mesh = plsc.VectorSubcoreMesh(core_axis_name="c", subcore_axis_name="s",
                                  num_cores=NC, num_subcores=NS)

    @functools.partial(
        pl.kernel,
        out_type=jax.ShapeDtypeStruct((NC, n_pad, d), jnp.float32),
        mesh=mesh,
        scratch_types=[
            pltpu.VMEM((chunks_per_tile, CHUNK), jnp.int32),
            pltpu.VMEM((CHUNK, d), jnp.float32),
            pltpu.VMEM_SHARED((n_pad, d), jnp.float32),
            pltpu.SemaphoreType.DMA,
        ],
    )
    def deg_k(edges_hbm, ones_hbm, zeros_hbm, out_hbm, idx_v, ones_v, deg_sh,
              sem):
        c = lax.axis_index("c")
        s = lax.axis_index("s")
        pltpu.sync_copy(ones_hbm, ones_v)
        pltpu.sync_copy(
            edges_hbm.at[c, pl.ds(s * chunks_per_tile, chunks_per_tile)],
            idx_v)
        pltpu.sync_copy(zeros_hbm, deg_sh.at[pl.ds(s * own, own)])
        plsc.subcore_barrier()

        def fire(j, carry):
            pltpu.async_copy(ones_v, deg_sh.at[idx_v.at[j]], sem, add=True)
            return carry

        lax.fori_loop(0, chunks_per_tile, fire, 0)

        def drain(j, carry):
            pltpu.make_async_copy(ones_v, deg_sh.at[idx_v.at[j]], sem).wait()
            return carry

        lax.fori_loop(0, chunks_per_tile, drain, 0)
        plsc.subcore_barrier()
        pltpu.sync_copy(deg_sh.at[pl.ds(s * own, own)],
                        out_hbm.at[c, pl.ds(s * own, own)])

    return deg_k


def _agg_kernel(n_pad, d, chunks_per_tile):
    own = n_pad // NS
    mesh = plsc.VectorSubcoreMesh(core_axis_name="c", subcore_axis_name="s",
                                  num_cores=NC, num_subcores=NS)

    half = chunks_per_tile // 2

    @functools.partial(
        pl.kernel,
        out_type=jax.ShapeDtypeStruct((NC, n_pad, d), jnp.float32),
        mesh=mesh,
        scratch_types=[
            pltpu.VMEM((half, CHUNK), jnp.int32),
            pltpu.VMEM((half, CHUNK), jnp.int32),
            pltpu.VMEM((CHUNK, d), jnp.float32),
            pltpu.VMEM((CHUNK, d), jnp.float32),
            pltpu.VMEM_SHARED((n_pad, d), jnp.float32),
            pltpu.SemaphoreType.DMA,
            pltpu.SemaphoreType.DMA,
        ],
    )
    def agg_k(edges_hbm, h_hbm, zeros_hbm, out_hbm,
              src_v, dst_v, buf_a, buf_b, agg_sh, sem_a, sem_b):
        c = lax.axis_index("c")
        s = lax.axis_index("s")
        w = c * NS + s
        pltpu.sync_copy(zeros_hbm, agg_sh.at[pl.ds(s * own, own)])
        plsc.subcore_barrier()

        npairs = half // 2
        for hh in range(2):
            base = w * chunks_per_tile + hh * half
            pltpu.sync_copy(edges_hbm.at[0, pl.ds(base, half)], src_v)
            pltpu.sync_copy(edges_hbm.at[1, pl.ds(base, half)], dst_v)
            pltpu.async_copy(h_hbm.at[src_v.at[0]], buf_a, sem_a)
            pltpu.async_copy(h_hbm.at[src_v.at[1]], buf_b, sem_b)

            def body(g, carry):
                ja = 2 * g
                jb = 2 * g + 1
                pltpu.make_async_copy(h_hbm.at[src_v.at[ja]], buf_a,
                                      sem_a).wait()
                pltpu.sync_copy(buf_a, agg_sh.at[dst_v.at[ja]], add=True)

                @pl.when(g + 1 < npairs)
                def _():
                    pltpu.async_copy(h_hbm.at[src_v.at[ja + 2]], buf_a, sem_a)

                pltpu.make_async_copy(h_hbm.at[src_v.at[jb]], buf_b,
                                      sem_b).wait()
                pltpu.sync_copy(buf_b, agg_sh.at[dst_v.at[jb]], add=True)

                @pl.when(g + 1 < npairs)
                def _():
                    pltpu.async_copy(h_hbm.at[src_v.at[jb + 2]], buf_b, sem_b)

                return carry

            lax.fori_loop(0, npairs, body, 0)
        plsc.subcore_barrier()
        pltpu.sync_copy(agg_sh.at[pl.ds(s * own, own)],
                        out_hbm.at[c, pl.ds(s * own, own)])

    return agg_k


def _scale_call(deg0, feats_pad, n_pad, d, blk):
    def k2(dg_ref, ft_ref, out_ref):
        norm = lax.rsqrt(jnp.maximum(dg_ref[...], 1.0))
        out_ref[...] = ft_ref[...] * norm

    return pl.pallas_call(
        k2,
        grid=(n_pad // blk,),
        in_specs=[pl.BlockSpec((blk, 1), lambda i: (i, 0)),
                  pl.BlockSpec((blk, d), lambda i: (i, 0))],
        out_specs=pl.BlockSpec((blk, d), lambda i: (i, 0)),
        out_shape=jax.ShapeDtypeStruct((n_pad, d), jnp.float32),
    )(deg0, feats_pad)


def _linear_call(p0, p1, deg1, feats_pad, W, b2, n, n_pad, d, blk):
    def k4a(p0_ref, p1_ref, dg_ref, ft_ref, w_ref, b_ref, h_ref, st_ref):
        i = pl.program_id(0)
        nd = lax.rsqrt(jnp.maximum(dg_ref[...], 1.0))
        agg = (p0_ref[...] + p1_ref[...]) * nd
        h = jnp.dot(agg, w_ref[...], preferred_element_type=jnp.float32)
        h = h + b_ref[...] + ft_ref[...]
        h_ref[...] = h
        rows = i * blk + lax.broadcasted_iota(jnp.int32, (blk, 1), 0)
        hm = jnp.where(rows < n, h, 0.0)
        s1 = jnp.sum(hm, axis=0, keepdims=True)
        s2 = jnp.sum(hm * hm, axis=0, keepdims=True)
        st = jnp.concatenate([s1, s2], axis=0)

        @pl.when(i == 0)
        def _():
            st_ref[...] = st

        @pl.when(i > 0)
        def _():
            st_ref[...] = st_ref[...] + st

    return pl.pallas_call(
        k4a,
        grid=(n_pad // blk,),
        in_specs=[pl.BlockSpec((blk, d), lambda i: (i, 0)),
                  pl.BlockSpec((blk, d), lambda i: (i, 0)),
                  pl.BlockSpec((blk, 1), lambda i: (i, 0)),
                  pl.BlockSpec((blk, d), lambda i: (i, 0)),
                  pl.BlockSpec((d, d), lambda i: (0, 0)),
                  pl.BlockSpec((1, d), lambda i: (0, 0))],
        out_specs=[pl.BlockSpec((blk, d), lambda i: (i, 0)),
                   pl.BlockSpec((2, d), lambda i: (0, 0))],
        out_shape=[jax.ShapeDtypeStruct((n_pad, d), jnp.float32),
                   jax.ShapeDtypeStruct((2, d), jnp.float32)],
    )(p0, p1, deg1, feats_pad, W, b2)


def _bn_call(h, stats, gamma2, beta2, n, n_pad, d, blk):
    def k4b(h_ref, st_ref, g_ref, bt_ref, out_ref):
        inv_n = 1.0 / n
        mean = st_ref[0:1, :] * inv_n
        var = st_ref[1:2, :] * inv_n - mean * mean
        inv = lax.rsqrt(var + 1e-5)
        out_ref[...] = (h_ref[...] - mean) * (inv * g_ref[...]) + bt_ref[...]

    return pl.pallas_call(
        k4b,
        grid=(n_pad // blk,),
        in_specs=[pl.BlockSpec((blk, d), lambda i: (i, 0)),
                  pl.BlockSpec((2, d), lambda i: (0, 0)),
                  pl.BlockSpec((1, d), lambda i: (0, 0)),
                  pl.BlockSpec((1, d), lambda i: (0, 0))],
        out_specs=pl.BlockSpec((blk, d), lambda i: (i, 0)),
        out_shape=jax.ShapeDtypeStruct((n_pad, d), jnp.float32),
    )(h, stats, gamma2, beta2)


@jax.jit
def kernel(feats, edge_index, W, b, gamma, beta):
    n, d = feats.shape
    e = edge_index.shape[1]

    e_unit = NW * CHUNK * 8
    e_pad = ((e + e_unit - 1) // e_unit) * e_unit
    n_unit = 8 * NS
    n_pad = ((n + 1 + n_unit - 1) // n_unit) * n_unit

    sent = n + jnp.arange(e_pad - e, dtype=jnp.int32) % (n_pad - n)
    edges = jnp.concatenate(
        [edge_index, jnp.stack([sent, sent])], axis=1)
    edges = edges.reshape(2, e_pad // CHUNK, CHUNK)

    ones_rows = jnp.ones((CHUNK, d), dtype=jnp.float32)
    zeros_agg = jnp.zeros((n_pad // NS, d), dtype=jnp.float32)
    feats_pad = jnp.concatenate(
        [feats, jnp.zeros((n_pad - n, d), dtype=feats.dtype)], axis=0)

    k1_chunks = e_pad // (NS * CHUNK)
    k3_chunks = e_pad // (NW * CHUNK)

    deg = _deg_kernel(n_pad, d, k1_chunks)(edges, ones_rows, zeros_agg)
    deg0 = deg[0, :, 0:1]
    deg1 = deg[1, :, 0:1]

    blk = n_pad // 4
    h_scaled = _scale_call(deg0, feats_pad, n_pad, d, blk)

    parts = _agg_kernel(n_pad, d, k3_chunks)(edges, h_scaled, zeros_agg)

    b2 = b.reshape(1, d)
    gamma2 = gamma.reshape(1, d)
    beta2 = beta.reshape(1, d)
    h, stats = _linear_call(parts[0], parts[1], deg1, feats_pad, W, b2,
                            n, n_pad, d, blk)
    out = _bn_call(h, stats, gamma2, beta2, n, n_pad, d, blk)
    return out[:n]


# speedup vs baseline: 8.3420x; 1.0435x over previous
import functools

import jax
import jax.numpy as jnp
from jax import lax
from jax.experimental import pallas as pl
from jax.experimental.pallas import tpu as pltpu
from jax.experimental.pallas import tpu_sc as plsc

NC = 2
NS = 16
L = 16
NW = NC * NS
CHUNK = 128


def _deg_kernel(n_pad, d, chunks_per_tile):
    own = n_pad // NS
    mesh = plsc.VectorSubcoreMesh(core_axis_name="c", subcore_axis_name="s",
                                  num_cores=NC, num_subcores=NS)

    @functools.partial(
        pl.kernel,
        out_type=jax.ShapeDtypeStruct((NC, n_pad, d), jnp.float32),
        mesh=mesh,
        scratch_types=[
            pltpu.VMEM((chunks_per_tile, CHUNK), jnp.int32),
            pltpu.VMEM((CHUNK, d), jnp.float32),
            pltpu.VMEM_SHARED((n_pad, d), jnp.float32),
            pltpu.SemaphoreType.DMA,
        ],
    )
    def deg_k(edges_hbm, ones_hbm, zeros_hbm, out_hbm, idx_v, ones_v, deg_sh,
              sem):
        c = lax.axis_index("c")
        s = lax.axis_index("s")
        pltpu.sync_copy(ones_hbm, ones_v)
        pltpu.sync_copy(
            edges_hbm.at[c, pl.ds(s * chunks_per_tile, chunks_per_tile)],
            idx_v)
        pltpu.sync_copy(zeros_hbm, deg_sh.at[pl.ds(s * own, own)])
        plsc.subcore_barrier()

        def fire(j, carry):
            pltpu.async_copy(ones_v, deg_sh.at[idx_v.at[j]], sem, add=True)
            return carry

        lax.fori_loop(0, chunks_per_tile, fire, 0)

        def drain(j, carry):
            pltpu.make_async_copy(ones_v, deg_sh.at[idx_v.at[j]], sem).wait()
            return carry

        lax.fori_loop(0, chunks_per_tile, drain, 0)
        plsc.subcore_barrier()
        pltpu.sync_copy(deg_sh.at[pl.ds(s * own, own)],
                        out_hbm.at[c, pl.ds(s * own, own)])

    return deg_k


def _agg_kernel(n_pad, d, chunks_per_tile):
    own = n_pad // NS
    mesh = plsc.VectorSubcoreMesh(core_axis_name="c", subcore_axis_name="s",
                                  num_cores=NC, num_subcores=NS)

    half = chunks_per_tile // 2

    @functools.partial(
        pl.kernel,
        out_type=jax.ShapeDtypeStruct((NC, n_pad, d), jnp.float32),
        mesh=mesh,
        scratch_types=[
            pltpu.VMEM((half, CHUNK), jnp.int32),
            pltpu.VMEM((half, CHUNK), jnp.int32),
            pltpu.VMEM((CHUNK, d), jnp.float32),
            pltpu.VMEM((CHUNK, d), jnp.float32),
            pltpu.VMEM_SHARED((n_pad, d), jnp.float32),
            pltpu.SemaphoreType.DMA,
            pltpu.SemaphoreType.DMA,
        ],
    )
    def agg_k(edges_hbm, h_hbm, zeros_hbm, out_hbm,
              src_v, dst_v, buf_a, buf_b, agg_sh, sem_a, sem_b):
        c = lax.axis_index("c")
        s = lax.axis_index("s")
        w = c * NS + s
        pltpu.sync_copy(zeros_hbm, agg_sh.at[pl.ds(s * own, own)])
        plsc.subcore_barrier()

        npairs = half // 2
        for hh in range(2):
            base = w * chunks_per_tile + hh * half
            pltpu.sync_copy(edges_hbm.at[0, pl.ds(base, half)], src_v)
            pltpu.sync_copy(edges_hbm.at[1, pl.ds(base, half)], dst_v)
            pltpu.async_copy(h_hbm.at[src_v.at[0]], buf_a, sem_a)
            pltpu.async_copy(h_hbm.at[src_v.at[1]], buf_b, sem_b)

            def body(g, carry):
                ja = 2 * g
                jb = 2 * g + 1
                pltpu.make_async_copy(h_hbm.at[src_v.at[ja]], buf_a,
                                      sem_a).wait()
                pltpu.sync_copy(buf_a, agg_sh.at[dst_v.at[ja]], add=True)

                @pl.when(g + 1 < npairs)
                def _():
                    pltpu.async_copy(h_hbm.at[src_v.at[ja + 2]], buf_a, sem_a)

                pltpu.make_async_copy(h_hbm.at[src_v.at[jb]], buf_b,
                                      sem_b).wait()
                pltpu.sync_copy(buf_b, agg_sh.at[dst_v.at[jb]], add=True)

                @pl.when(g + 1 < npairs)
                def _():
                    pltpu.async_copy(h_hbm.at[src_v.at[jb + 2]], buf_b, sem_b)

                return carry

            lax.fori_loop(0, npairs, body, 0)
        plsc.subcore_barrier()
        pltpu.sync_copy(agg_sh.at[pl.ds(s * own, own)],
                        out_hbm.at[c, pl.ds(s * own, own)])

    return agg_k


def _scale_call(deg, feats, n_pad, d, blk):
    def k2(dg_ref, ft_ref, out_ref):
        norm = lax.rsqrt(jnp.maximum(dg_ref[0, :, 0:1], 1.0))
        out_ref[...] = ft_ref[...] * norm

    return pl.pallas_call(
        k2,
        grid=(n_pad // blk,),
        in_specs=[pl.BlockSpec((1, blk, d), lambda i: (0, i, 0)),
                  pl.BlockSpec((blk, d), lambda i: (i, 0))],
        out_specs=pl.BlockSpec((blk, d), lambda i: (i, 0)),
        out_shape=jax.ShapeDtypeStruct((n_pad, d), jnp.float32),
    )(deg, feats)


def _linear_call(p0, p1, deg, feats, W, b2, n, n_pad, d, blk):
    def k4a(p0_ref, p1_ref, dg_ref, ft_ref, w_ref, b_ref, h_ref, st_ref):
        i = pl.program_id(0)
        nd = lax.rsqrt(jnp.maximum(dg_ref[0, :, 0:1], 1.0))
        agg = (p0_ref[...] + p1_ref[...]) * nd
        h = jnp.dot(agg, w_ref[...], preferred_element_type=jnp.float32)
        h = h + b_ref[...] + ft_ref[...]
        h_ref[...] = h
        rows = i * blk + lax.broadcasted_iota(jnp.int32, (blk, 1), 0)
        hm = jnp.where(rows < n, h, 0.0)
        s1 = jnp.sum(hm, axis=0, keepdims=True)
        s2 = jnp.sum(hm * hm, axis=0, keepdims=True)
        st = jnp.concatenate([s1, s2], axis=0)

        @pl.when(i == 0)
        def _():
            st_ref[...] = st

        @pl.when(i > 0)
        def _():
            st_ref[...] = st_ref[...] + st

    return pl.pallas_call(
        k4a,
        grid=(n_pad // blk,),
        in_specs=[pl.BlockSpec((blk, d), lambda i: (i, 0)),
                  pl.BlockSpec((blk, d), lambda i: (i, 0)),
                  pl.BlockSpec((1, blk, d), lambda i: (1, i, 0)),
                  pl.BlockSpec((blk, d), lambda i: (i, 0)),
                  pl.BlockSpec((d, d), lambda i: (0, 0)),
                  pl.BlockSpec((1, d), lambda i: (0, 0))],
        out_specs=[pl.BlockSpec((blk, d), lambda i: (i, 0)),
                   pl.BlockSpec((2, d), lambda i: (0, 0))],
        out_shape=[jax.ShapeDtypeStruct((n_pad, d), jnp.float32),
                   jax.ShapeDtypeStruct((2, d), jnp.float32)],
    )(p0, p1, deg, feats, W, b2)


def _bn_call(h, stats, gamma2, beta2, n, n_pad, d, blk):
    def k4b(h_ref, st_ref, g_ref, bt_ref, out_ref):
        inv_n = 1.0 / n
        mean = st_ref[0:1, :] * inv_n
        var = st_ref[1:2, :] * inv_n - mean * mean
        inv = lax.rsqrt(var + 1e-5)
        out_ref[...] = (h_ref[...] - mean) * (inv * g_ref[...]) + bt_ref[...]

    return pl.pallas_call(
        k4b,
        grid=(n_pad // blk,),
        in_specs=[pl.BlockSpec((blk, d), lambda i: (i, 0)),
                  pl.BlockSpec((2, d), lambda i: (0, 0)),
                  pl.BlockSpec((1, d), lambda i: (0, 0)),
                  pl.BlockSpec((1, d), lambda i: (0, 0))],
        out_specs=pl.BlockSpec((blk, d), lambda i: (i, 0)),
        out_shape=jax.ShapeDtypeStruct((n, d), jnp.float32),
    )(h, stats, gamma2, beta2)


@jax.jit
def kernel(feats, edge_index, W, b, gamma, beta):
    n, d = feats.shape
    e = edge_index.shape[1]

    e_unit = NW * CHUNK * 8
    e_pad = ((e + e_unit - 1) // e_unit) * e_unit
    n_unit = 8 * NS
    n_pad = ((n + 1 + n_unit - 1) // n_unit) * n_unit

    sent = n + jnp.arange(e_pad - e, dtype=jnp.int32) % (n_pad - n)
    edges = jnp.concatenate(
        [edge_index, jnp.stack([sent, sent])], axis=1)
    edges = edges.reshape(2, e_pad // CHUNK, CHUNK)

    ones_rows = jnp.ones((CHUNK, d), dtype=jnp.float32)
    zeros_agg = jnp.zeros((n_pad // NS, d), dtype=jnp.float32)

    k1_chunks = e_pad // (NS * CHUNK)
    k3_chunks = e_pad // (NW * CHUNK)

    deg = _deg_kernel(n_pad, d, k1_chunks)(edges, ones_rows, zeros_agg)

    blk = n_pad // 4
    h_scaled = _scale_call(deg, feats, n_pad, d, blk)

    parts = _agg_kernel(n_pad, d, k3_chunks)(edges, h_scaled, zeros_agg)

    b2 = b.reshape(1, d)
    gamma2 = gamma.reshape(1, d)
    beta2 = beta.reshape(1, d)
    h, stats = _linear_call(parts[0], parts[1], deg, feats, W, b2,
                            n, n_pad, d, blk)
    return _bn_call(h, stats, gamma2, beta2, n, n_pad, d, blk)


# speedup vs baseline: 8.6793x; 1.0404x over previous
import functools

import numpy as np
import jax
import jax.numpy as jnp
from jax import lax
from jax.experimental import pallas as pl
from jax.experimental.pallas import tpu as pltpu
from jax.experimental.pallas import tpu_sc as plsc

NC = 2
NS = 16
NW = NC * NS
CHUNK = 128


def _split(chunks, parts):
    per = -(-chunks // parts)
    per = ((per + 7) // 8) * 8
    full = chunks // per
    tail = chunks - full * per
    return per, full, tail


def _deg_kernel(n_pad, d, chunks):
    own = n_pad // NS
    per, full, tail = _split(chunks, NS)
    mesh = plsc.VectorSubcoreMesh(core_axis_name="c", subcore_axis_name="s",
                                  num_cores=NC, num_subcores=NS)

    @functools.partial(
        pl.kernel,
        out_type=jax.ShapeDtypeStruct((NC, n_pad, d), jnp.float32),
        mesh=mesh,
        scratch_types=[
            pltpu.VMEM((per, CHUNK), jnp.int32),
            pltpu.VMEM((CHUNK, d), jnp.float32),
            pltpu.VMEM_SHARED((n_pad, d), jnp.float32),
            pltpu.SemaphoreType.DMA,
        ],
    )
    def deg_k(edges_hbm, ones_hbm, zeros_hbm, out_hbm, idx_v, ones_v, deg_sh,
              sem):
        c = lax.axis_index("c")
        s = lax.axis_index("s")
        pltpu.sync_copy(ones_hbm, ones_v)

        @pl.when(s < full)
        def _():
            pltpu.sync_copy(edges_hbm.at[c, pl.ds(s * per, per)], idx_v)

        if tail:
            @pl.when(s == full)
            def _():
                pltpu.sync_copy(edges_hbm.at[c, pl.ds(full * per, tail)],
                                idx_v.at[pl.ds(0, tail)])

        cnt = jnp.where(s < full, per, jnp.where(s == full, tail, 0))
        pltpu.sync_copy(zeros_hbm, deg_sh.at[pl.ds(s * own, own)])
        plsc.subcore_barrier()

        def fire(j, carry):
            pltpu.async_copy(ones_v, deg_sh.at[idx_v.at[j]], sem, add=True)
            return carry

        lax.fori_loop(0, cnt, fire, 0)

        def drain(j, carry):
            pltpu.make_async_copy(ones_v, deg_sh.at[idx_v.at[j]], sem).wait()
            return carry

        lax.fori_loop(0, cnt, drain, 0)
        plsc.subcore_barrier()
        pltpu.sync_copy(deg_sh.at[pl.ds(s * own, own)],
                        out_hbm.at[c, pl.ds(s * own, own)])

    return deg_k


def _agg_kernel(n_pad, d, chunks):
    own = n_pad // NS
    per, full, tail = _split(chunks, NW)
    stage = per // 2
    mesh = plsc.VectorSubcoreMesh(core_axis_name="c", subcore_axis_name="s",
                                  num_cores=NC, num_subcores=NS)

    @functools.partial(
        pl.kernel,
        out_type=jax.ShapeDtypeStruct((NC, n_pad, d), jnp.float32),
        mesh=mesh,
        scratch_types=[
            pltpu.VMEM((stage, CHUNK), jnp.int32),
            pltpu.VMEM((stage, CHUNK), jnp.int32),
            pltpu.VMEM((CHUNK, d), jnp.float32),
            pltpu.VMEM((CHUNK, d), jnp.float32),
            pltpu.VMEM_SHARED((n_pad, d), jnp.float32),
            pltpu.SemaphoreType.DMA,
            pltpu.SemaphoreType.DMA,
        ],
    )
    def agg_k(edges_hbm, h_hbm, zeros_hbm, out_hbm,
              src_v, dst_v, buf_a, buf_b, agg_sh, sem_a, sem_b):
        c = lax.axis_index("c")
        s = lax.axis_index("s")
        w = c * NS + s
        pltpu.sync_copy(zeros_hbm, agg_sh.at[pl.ds(s * own, own)])
        plsc.subcore_barrier()

        def run_stage(base, sz):
            npairs = sz // 2
            pltpu.sync_copy(edges_hbm.at[0, pl.ds(base, sz)],
                            src_v.at[pl.ds(0, sz)])
            pltpu.sync_copy(edges_hbm.at[1, pl.ds(base, sz)],
                            dst_v.at[pl.ds(0, sz)])
            pltpu.async_copy(h_hbm.at[src_v.at[0]], buf_a, sem_a)
            pltpu.async_copy(h_hbm.at[src_v.at[1]], buf_b, sem_b)

            def body(g, carry):
                ja = 2 * g
                jb = 2 * g + 1
                pltpu.make_async_copy(h_hbm.at[src_v.at[ja]], buf_a,
                                      sem_a).wait()
                pltpu.sync_copy(buf_a, agg_sh.at[dst_v.at[ja]], add=True)

                @pl.when(g + 1 < npairs)
                def _():
                    pltpu.async_copy(h_hbm.at[src_v.at[ja + 2]], buf_a, sem_a)

                pltpu.make_async_copy(h_hbm.at[src_v.at[jb]], buf_b,
                                      sem_b).wait()
                pltpu.sync_copy(buf_b, agg_sh.at[dst_v.at[jb]], add=True)

                @pl.when(g + 1 < npairs)
                def _():
                    pltpu.async_copy(h_hbm.at[src_v.at[jb + 2]], buf_b, sem_b)

                return carry

            lax.fori_loop(0, npairs, body, 0)

        @pl.when(w < full)
        def _():
            run_stage(w * per, stage)
            run_stage(w * per + stage, stage)

        if tail:
            @pl.when(w == full)
            def _():
                run_stage(full * per, tail)

        plsc.subcore_barrier()
        pltpu.sync_copy(agg_sh.at[pl.ds(s * own, own)],
                        out_hbm.at[c, pl.ds(s * own, own)])

    return agg_k


def _scale_call(deg, feats, n_pad, d, blk):
    def k2(dg_ref, ft_ref, out_ref):
        norm = lax.rsqrt(jnp.maximum(dg_ref[0, :, 0:1], 1.0))
        out_ref[...] = ft_ref[...] * norm

    return pl.pallas_call(
        k2,
        grid=(n_pad // blk,),
        in_specs=[pl.BlockSpec((1, blk, d), lambda i: (0, i, 0)),
                  pl.BlockSpec((blk, d), lambda i: (i, 0))],
        out_specs=pl.BlockSpec((blk, d), lambda i: (i, 0)),
        out_shape=jax.ShapeDtypeStruct((n_pad, d), jnp.float32),
    )(deg, feats)


def _linear_call(parts, deg, feats, W, b2, n, n_pad, d, blk):
    def k4a(pp_ref, dg_ref, ft_ref, w_ref, b_ref, h_ref, st_ref):
        i = pl.program_id(0)
        nd = lax.rsqrt(jnp.maximum(dg_ref[0, :, 0:1], 1.0))
        agg = (pp_ref[0] + pp_ref[1]) * nd
        h = jnp.dot(agg, w_ref[...], preferred_element_type=jnp.float32)
        h = h + b_ref[...] + ft_ref[...]
        h_ref[...] = h
        rows = i * blk + lax.broadcasted_iota(jnp.int32, (blk, 1), 0)
        hm = jnp.where(rows < n, h, 0.0)
        s1 = jnp.sum(hm, axis=0, keepdims=True)
        s2 = jnp.sum(hm * hm, axis=0, keepdims=True)
        st = jnp.concatenate([s1, s2], axis=0)

        @pl.when(i == 0)
        def _():
            st_ref[...] = st

        @pl.when(i > 0)
        def _():
            st_ref[...] = st_ref[...] + st

    return pl.pallas_call(
        k4a,
        grid=(n_pad // blk,),
        in_specs=[pl.BlockSpec((2, blk, d), lambda i: (0, i, 0)),
                  pl.BlockSpec((1, blk, d), lambda i: (1, i, 0)),
                  pl.BlockSpec((blk, d), lambda i: (i, 0)),
                  pl.BlockSpec((d, d), lambda i: (0, 0)),
                  pl.BlockSpec((1, d), lambda i: (0, 0))],
        out_specs=[pl.BlockSpec((blk, d), lambda i: (i, 0)),
                   pl.BlockSpec((2, d), lambda i: (0, 0))],
        out_shape=[jax.ShapeDtypeStruct((n_pad, d), jnp.float32),
                   jax.ShapeDtypeStruct((2, d), jnp.float32)],
    )(parts, deg, feats, W, b2)


def _bn_call(h, stats, gamma2, beta2, n, n_pad, d, blk):
    def k4b(h_ref, st_ref, g_ref, bt_ref, out_ref):
        inv_n = 1.0 / n
        mean = st_ref[0:1, :] * inv_n
        var = st_ref[1:2, :] * inv_n - mean * mean
        inv = lax.rsqrt(var + 1e-5)
        out_ref[...] = (h_ref[...] - mean) * (inv * g_ref[...]) + bt_ref[...]

    return pl.pallas_call(
        k4b,
        grid=(n_pad // blk,),
        in_specs=[pl.BlockSpec((blk, d), lambda i: (i, 0)),
                  pl.BlockSpec((2, d), lambda i: (0, 0)),
                  pl.BlockSpec((1, d), lambda i: (0, 0)),
                  pl.BlockSpec((1, d), lambda i: (0, 0))],
        out_specs=pl.BlockSpec((blk, d), lambda i: (i, 0)),
        out_shape=jax.ShapeDtypeStruct((n, d), jnp.float32),
    )(h, stats, gamma2, beta2)


@jax.jit
def kernel(feats, edge_index, W, b, gamma, beta):
    n, d = feats.shape
    e = edge_index.shape[1]

    n_unit = 8 * NS
    n_pad = ((n + 1 + n_unit - 1) // n_unit) * n_unit

    if e % CHUNK:
        pad = CHUNK - e % CHUNK
        sent = n + jnp.arange(pad, dtype=jnp.int32) % (n_pad - n)
        edge_index = jnp.concatenate([edge_index, jnp.stack([sent, sent])],
                                     axis=1)
        e += pad
    chunks = e // CHUNK
    edges = edge_index.reshape(2, chunks, CHUNK)

    ones_rows = jnp.asarray(np.ones((CHUNK, d), dtype=np.float32))
    zeros_agg = jnp.asarray(np.zeros((n_pad // NS, d), dtype=np.float32))

    deg = _deg_kernel(n_pad, d, chunks)(edges, ones_rows, zeros_agg)

    blk = n_pad // 4
    h_scaled = _scale_call(deg, feats, n_pad, d, blk)

    parts = _agg_kernel(n_pad, d, chunks)(edges, h_scaled, zeros_agg)

    b2 = b.reshape(1, d)
    gamma2 = gamma.reshape(1, d)
    beta2 = beta.reshape(1, d)
    h, stats = _linear_call(parts, deg, feats, W, b2, n, n_pad, d, blk)
    return _bn_call(h, stats, gamma2, beta2, n, n_pad, d, blk)
